# Initial kernel scaffold; baseline (speedup 1.0000x reference)
import jax
import jax.numpy as jnp
from jax.experimental import pallas as pl


def kernel(idx, emb_weight):
    raise NotImplementedError("write your pallas kernel here")


# speedup vs baseline: 1.9385x; 1.9385x over previous
import functools

import jax
import jax.numpy as jnp
from jax import lax
from jax.experimental import pallas as pl
from jax.experimental.pallas import tpu as pltpu
from jax.experimental.pallas import tpu_sc as plsc

VOCAB = 8192
N_IDX = 4 * 2048
NC = 2
NS = 16
NW = NC * NS
PER_W = N_IDX // NW
W = 2
NBUF = 4
NWIN = PER_W // W
NGRP = NWIN // NBUF

_mesh = plsc.VectorSubcoreMesh(core_axis_name="c", subcore_axis_name="s")


@functools.partial(
    pl.kernel,
    out_type=jax.ShapeDtypeStruct((N_IDX, VOCAB), jnp.float32),
    mesh=_mesh,
    scratch_types=[
        pltpu.VMEM((NWIN, W), jnp.int32),
        pltpu.VMEM((NBUF, W, VOCAB), jnp.float32),
        pltpu.SemaphoreType.DMA((NBUF,)),
        pltpu.SemaphoreType.DMA((NBUF,)),
    ],
)
def _lookup(idx_hbm, table_hbm, out_hbm, idx_v, rows_v, gsem, osem):
    wid = lax.axis_index("s") * NC + lax.axis_index("c")
    row0 = wid * PER_W

    pltpu.sync_copy(idx_hbm.at[wid], idx_v)

    def g_start(w, b):
        pltpu.async_copy(table_hbm.at[idx_v.at[w]], rows_v.at[b], gsem.at[b])

    def g_wait(w, b):
        pltpu.make_async_copy(
            table_hbm.at[idx_v.at[w]], rows_v.at[b], gsem.at[b]
        ).wait()

    def o_start(w, b):
        pltpu.async_copy(
            rows_v.at[b], out_hbm.at[pl.ds(row0 + w * W, W)], osem.at[b]
        )

    def o_wait(w, b):
        pltpu.make_async_copy(
            rows_v.at[b], out_hbm.at[pl.ds(row0 + w * W, W)], osem.at[b]
        ).wait()

    g_start(0, 0)
    g_start(1, 1)
    for j in range(NBUF):
        b2 = (j + 2) % NBUF
        if j >= 2:
            o_wait(j - 2, b2)
        g_start(j + 2, b2)
        g_wait(j, j)
        o_start(j, j)

    def body(i, carry):
        for j in range(NBUF):
            w = i * NBUF + j
            b2 = (j + 2) % NBUF
            o_wait(w - 2, b2)
            g_start(w + 2, b2)
            g_wait(w, j)
            o_start(w, j)
        return carry

    lax.fori_loop(1, NGRP - 1, body, 0)

    for j in range(NBUF):
        w = (NGRP - 1) * NBUF + j
        b2 = (j + 2) % NBUF
        o_wait(w - 2, b2)
        if w + 2 < NWIN:
            g_start(w + 2, b2)
        g_wait(w, j)
        o_start(w, j)

    for j in range(2, NBUF):
        w = NWIN - NBUF + j
        o_wait(w, j)


def kernel(idx, emb_weight):
    idx3 = idx.reshape(NW, NWIN, W)
    out = _lookup(idx3, emb_weight)
    return out.reshape(idx.shape[0], idx.shape[1], VOCAB)
